# Initial kernel scaffold; baseline (speedup 1.0000x reference)
import jax
import jax.numpy as jnp
from jax.experimental import pallas as pl


def kernel(input_x, table, enc_w, enc_b, dec_w, dec_b):
    raise NotImplementedError("write your pallas kernel here")


# speedup vs baseline: 1.0708x; 1.0708x over previous
import functools

import jax
import jax.numpy as jnp
from jax import lax
from jax.experimental import pallas as pl
from jax.experimental.pallas import tpu as pltpu
from jax.experimental.pallas import tpu_sc as plsc


def _encode_table(table, enc_w, enc_b):
    V, D = table.shape
    C = enc_w.shape[1]
    BLK = 8000
    assert V % BLK == 0

    def body(t_ref, w_ref, b_ref, o_ref):
        acc = jnp.dot(t_ref[...], w_ref[...], preferred_element_type=jnp.float32)
        o_ref[...] = jnp.maximum(acc + b_ref[...], 0.0)

    return pl.pallas_call(
        body,
        grid=(V // BLK,),
        in_specs=[
            pl.BlockSpec((BLK, D), lambda i: (i, 0)),
            pl.BlockSpec((D, C), lambda i: (0, 0)),
            pl.BlockSpec((1, C), lambda i: (0, 0)),
        ],
        out_specs=pl.BlockSpec((BLK, C), lambda i: (i, 0)),
        out_shape=jax.ShapeDtypeStruct((V, C), jnp.float32),
    )(table, enc_w, enc_b.reshape(1, C))


def _sc_gather(te, idx_flat):
    N = idx_flat.shape[0]
    C = te.shape[1]
    info = plsc.get_sparse_core_info()
    NC, NS = info.num_cores, info.num_subcores
    NW = NC * NS
    assert N % NW == 0
    b_per_w = N // NW
    CH = 2560
    assert b_per_w % CH == 0
    n_ch = b_per_w // CH
    mesh = plsc.VectorSubcoreMesh(core_axis_name="c", subcore_axis_name="s")

    @functools.partial(
        pl.kernel,
        out_type=jax.ShapeDtypeStruct((N, C), jnp.float32),
        mesh=mesh,
        compiler_params=pltpu.CompilerParams(use_tc_tiling_on_sc=False),
        scratch_types=[
            pltpu.VMEM((CH,), jnp.int32),
            pltpu.VMEM((CH, C), jnp.float32),
            pltpu.SemaphoreType.DMA,
        ],
    )
    def k(idx_hbm, te_hbm, out_hbm, idx_v, rows_v, sem):
        wid = lax.axis_index("s") * NC + lax.axis_index("c")
        base = wid * b_per_w

        def body(c, carry):
            off = base + c * CH
            pltpu.sync_copy(idx_hbm.at[pl.ds(off, CH)], idx_v)
            pltpu.async_copy(te_hbm.at[idx_v], rows_v, sem).wait()
            pltpu.sync_copy(rows_v, out_hbm.at[pl.ds(off, CH)])
            return carry

        lax.fori_loop(0, n_ch, body, 0)

    return k(idx_flat, te)


def _decode(g, dec_w, dec_b):
    N, C = g.shape
    D = dec_w.shape[1]
    BLK = 8192
    assert N % BLK == 0

    def body(g_ref, w_ref, b_ref, o_ref):
        acc = jnp.dot(g_ref[...], w_ref[...], preferred_element_type=jnp.float32)
        o_ref[...] = jnp.maximum(acc + b_ref[...], 0.0)

    return pl.pallas_call(
        body,
        grid=(N // BLK,),
        in_specs=[
            pl.BlockSpec((BLK, C), lambda i: (i, 0)),
            pl.BlockSpec((C, D), lambda i: (0, 0)),
            pl.BlockSpec((1, D), lambda i: (0, 0)),
        ],
        out_specs=pl.BlockSpec((BLK, D), lambda i: (i, 0)),
        out_shape=jax.ShapeDtypeStruct((N, D), jnp.float32),
    )(g, dec_w, dec_b.reshape(1, D))


def kernel(input_x, table, enc_w, enc_b, dec_w, dec_b):
    B, L = input_x.shape
    D = table.shape[1]
    idx_flat = input_x.reshape(-1).astype(jnp.int32)
    te = _encode_table(table, enc_w, enc_b)
    g = _sc_gather(te, idx_flat)
    out = _decode(g, dec_w, dec_b)
    return out.reshape(B, L, D)


# speedup vs baseline: 1.0790x; 1.0076x over previous
import functools

import jax
import jax.numpy as jnp
from jax import lax
from jax.experimental import pallas as pl
from jax.experimental.pallas import tpu as pltpu
from jax.experimental.pallas import tpu_sc as plsc


def _encode_table(table, enc_w, enc_b):
    V, D = table.shape
    C = enc_w.shape[1]
    BLK = 8000
    assert V % BLK == 0

    def body(t_ref, w_ref, b_ref, o_ref):
        acc = jnp.dot(t_ref[...], w_ref[...], preferred_element_type=jnp.float32)
        o_ref[...] = jnp.maximum(acc + b_ref[...], 0.0)

    return pl.pallas_call(
        body,
        grid=(V // BLK,),
        in_specs=[
            pl.BlockSpec((BLK, D), lambda i: (i, 0)),
            pl.BlockSpec((D, C), lambda i: (0, 0)),
            pl.BlockSpec((1, C), lambda i: (0, 0)),
        ],
        out_specs=pl.BlockSpec((BLK, C), lambda i: (i, 0)),
        out_shape=jax.ShapeDtypeStruct((V, C), jnp.float32),
    )(table, enc_w, enc_b.reshape(1, C))


def _sc_gather(te, idx_flat):
    N = idx_flat.shape[0]
    C = te.shape[1]
    info = plsc.get_sparse_core_info()
    NC, NS = info.num_cores, info.num_subcores
    NW = NC * NS
    assert N % NW == 0
    b_per_w = N // NW
    CH = 1600
    NBUF = 3
    assert b_per_w % CH == 0
    n_ch = b_per_w // CH
    assert n_ch > NBUF
    mesh = plsc.VectorSubcoreMesh(core_axis_name="c", subcore_axis_name="s")

    @functools.partial(
        pl.kernel,
        out_type=jax.ShapeDtypeStruct((N, C), jnp.float32),
        mesh=mesh,
        compiler_params=pltpu.CompilerParams(use_tc_tiling_on_sc=False),
        scratch_types=[
            pltpu.VMEM((NBUF, CH), jnp.int32),
            pltpu.VMEM((NBUF, CH, C), jnp.float32),
            pltpu.SemaphoreType.DMA((NBUF,)),
            pltpu.SemaphoreType.DMA((NBUF,)),
            pltpu.SemaphoreType.DMA((NBUF,)),
        ],
    )
    def k(idx_hbm, te_hbm, out_hbm, idx_v, rows_v, is_sem, gs_sem, os_sem):
        wid = lax.axis_index("s") * NC + lax.axis_index("c")
        base = wid * b_per_w

        def idx_copy(c):
            b = c % NBUF
            return pltpu.make_async_copy(
                idx_hbm.at[pl.ds(base + c * CH, CH)], idx_v.at[b], is_sem.at[b])

        def gat_copy(c):
            b = c % NBUF
            return pltpu.make_async_copy(
                te_hbm.at[idx_v.at[b]], rows_v.at[b], gs_sem.at[b])

        def out_copy(c):
            b = c % NBUF
            return pltpu.make_async_copy(
                rows_v.at[b], out_hbm.at[pl.ds(base + c * CH, CH)], os_sem.at[b])

        for c in range(NBUF):
            idx_copy(c).start()
        idx_copy(0).wait()
        gat_copy(0).start()
        idx_copy(1).wait()
        gat_copy(1).start()
        for c in range(n_ch):
            gat_copy(c).wait()
            if c + NBUF < n_ch:
                idx_copy(c + NBUF).start()
            out_copy(c).start()
            if c + 2 < n_ch:
                idx_copy(c + 2).wait()
                if c >= 1:
                    out_copy(c - 1).wait()
                gat_copy(c + 2).start()
        for c in range(n_ch - 3, n_ch):
            out_copy(c).wait()

    return k(idx_flat, te)


def _decode(g, dec_w, dec_b):
    N, C = g.shape
    D = dec_w.shape[1]
    BLK = 8192
    assert N % BLK == 0

    def body(g_ref, w_ref, b_ref, o_ref):
        acc = jnp.dot(g_ref[...], w_ref[...], preferred_element_type=jnp.float32)
        o_ref[...] = jnp.maximum(acc + b_ref[...], 0.0)

    return pl.pallas_call(
        body,
        grid=(N // BLK,),
        in_specs=[
            pl.BlockSpec((BLK, C), lambda i: (i, 0)),
            pl.BlockSpec((C, D), lambda i: (0, 0)),
            pl.BlockSpec((1, D), lambda i: (0, 0)),
        ],
        out_specs=pl.BlockSpec((BLK, D), lambda i: (i, 0)),
        out_shape=jax.ShapeDtypeStruct((N, D), jnp.float32),
    )(g, dec_w, dec_b.reshape(1, D))


def kernel(input_x, table, enc_w, enc_b, dec_w, dec_b):
    B, L = input_x.shape
    D = table.shape[1]
    idx_flat = input_x.reshape(-1).astype(jnp.int32)
    te = _encode_table(table, enc_w, enc_b)
    g = _sc_gather(te, idx_flat)
    out = _decode(g, dec_w, dec_b)
    return out.reshape(B, L, D)


# speedup vs baseline: 2.2266x; 2.0636x over previous
import functools

import jax
import jax.numpy as jnp
from jax import lax
from jax.experimental import pallas as pl
from jax.experimental.pallas import tpu as pltpu
from jax.experimental.pallas import tpu_sc as plsc

V = 1000000
D = 64
C = 16
ENC_BLK = 4096


def _encode_body(t_ref, w_ref, b_ref, *o_refs):
    x = jnp.dot(w_ref[...], t_ref[...], preferred_element_type=jnp.float32)
    x = jnp.maximum(x + b_ref[...], 0.0)
    for c in range(C):
        o_refs[c][...] = x[c, :]


def _encode_table(tableT, enc_wT, enc_b):
    ng = (V + ENC_BLK - 1) // ENC_BLK
    return pl.pallas_call(
        _encode_body,
        grid=(ng,),
        in_specs=[
            pl.BlockSpec((D, ENC_BLK), lambda i: (0, i)),
            pl.BlockSpec((C, D), lambda i: (0, 0)),
            pl.BlockSpec((C, 1), lambda i: (0, 0)),
        ],
        out_specs=[pl.BlockSpec((ENC_BLK,), lambda i: (i,)) for _ in range(C)],
        out_shape=[jax.ShapeDtypeStruct((V,), jnp.float32) for _ in range(C)],
    )(tableT, enc_wT, enc_b.reshape(C, 1))


PACK_CH = 2000
PACK_NCH = V // PACK_CH


def _sc_pack(te_cols):
    info = plsc.get_sparse_core_info()
    NC, NS = info.num_cores, info.num_subcores
    NW = NC * NS
    mesh = plsc.VectorSubcoreMesh(core_axis_name="c", subcore_axis_name="s")
    max_k = (PACK_NCH + NW - 1) // NW

    @functools.partial(
        pl.kernel,
        out_type=jax.ShapeDtypeStruct((V, C), jnp.float32),
        mesh=mesh,
        compiler_params=pltpu.CompilerParams(use_tc_tiling_on_sc=False, needs_layout_passes=False),
        scratch_types=[
            pltpu.VMEM((C, PACK_CH), jnp.float32),
            pltpu.VMEM((PACK_CH, C), jnp.float32),
        ],
    )
    def k(*refs):
        col_hbm = refs[:C]
        te_hbm = refs[C]
        in_v, out_v = refs[C + 1], refs[C + 2]
        wid = lax.axis_index("s") * NC + lax.axis_index("c")
        lanes = jnp.arange(16, dtype=jnp.int32)

        def do_chunk(g):
            off = g * PACK_CH
            for c in range(C):
                pltpu.sync_copy(col_hbm[c].at[pl.ds(off, PACK_CH)], in_v.at[c])

            def tpose(t8, carry):
                for u in range(8):
                    t = t8 * 8 + u
                    col = plsc.load_gather(in_v, [lanes, jnp.full((16,), t, jnp.int32)])
                    out_v[t, :] = col
                return carry

            lax.fori_loop(0, PACK_CH // 8, tpose, 0)
            pltpu.sync_copy(out_v, te_hbm.at[pl.ds(off, PACK_CH)])

        def body(kk, carry):
            g = wid + kk * NW

            @pl.when(g < PACK_NCH)
            def _():
                do_chunk(g)

            return carry

        lax.fori_loop(0, max_k, body, 0)

    return k(*te_cols)


GAT_CH = 1600
GAT_NBUF = 2


def _sc_gather_t(te, idx_flat):
    N = idx_flat.shape[0]
    info = plsc.get_sparse_core_info()
    NC, NS = info.num_cores, info.num_subcores
    NW = NC * NS
    assert N % NW == 0
    b_per_w = N // NW
    assert b_per_w % GAT_CH == 0
    n_ch = b_per_w // GAT_CH
    mesh = plsc.VectorSubcoreMesh(core_axis_name="c", subcore_axis_name="s")

    @functools.partial(
        pl.kernel,
        out_type=[jax.ShapeDtypeStruct((N,), jnp.float32) for _ in range(C)],
        mesh=mesh,
        compiler_params=pltpu.CompilerParams(use_tc_tiling_on_sc=False, needs_layout_passes=False),
        scratch_types=[
            pltpu.VMEM((GAT_NBUF, GAT_CH), jnp.int32),
            pltpu.VMEM((GAT_NBUF, GAT_CH, C), jnp.float32),
            pltpu.VMEM((GAT_NBUF, C, GAT_CH), jnp.float32),
            pltpu.SemaphoreType.DMA((GAT_NBUF,)),
            pltpu.SemaphoreType.DMA((GAT_NBUF,)),
            pltpu.SemaphoreType.DMA((GAT_NBUF,)),
        ],
    )
    def k(*refs):
        idx_hbm, te_hbm = refs[0], refs[1]
        g_hbm = refs[2:2 + C]
        idx_v, rows_v, trows_v = refs[2 + C], refs[3 + C], refs[4 + C]
        is_sem, gs_sem, os_sem = refs[5 + C], refs[6 + C], refs[7 + C]
        wid = lax.axis_index("s") * NC + lax.axis_index("c")
        base = wid * b_per_w
        lanes = jnp.arange(16, dtype=jnp.int32)

        def idx_copy(ch):
            b = ch % GAT_NBUF
            return pltpu.make_async_copy(
                idx_hbm.at[pl.ds(base + ch * GAT_CH, GAT_CH)], idx_v.at[b],
                is_sem.at[b])

        def gat_copy(ch):
            b = ch % GAT_NBUF
            return pltpu.make_async_copy(
                te_hbm.at[idx_v.at[b]], rows_v.at[b], gs_sem.at[b])

        def out_copy(ch, c):
            b = ch % GAT_NBUF
            return pltpu.make_async_copy(
                trows_v.at[b, c], g_hbm[c].at[pl.ds(base + ch * GAT_CH, GAT_CH)],
                os_sem.at[b])

        def tpose(ch):
            b = ch % GAT_NBUF

            def step(t8, carry):
                for u in range(8):
                    t = t8 * 8 + u
                    row = rows_v[b, t, :]
                    plsc.store_scatter(
                        trows_v.at[b], [lanes, jnp.full((16,), t, jnp.int32)], row)
                return carry

            lax.fori_loop(0, GAT_CH // 8, step, 0)

        idx_copy(0).start()
        idx_copy(1).start()
        idx_copy(0).wait()
        gat_copy(0).start()

        for ch in range(n_ch):
            gat_copy(ch).wait()
            if ch + 1 < n_ch:
                idx_copy(ch + 1).wait()
                gat_copy(ch + 1).start()
            if ch >= GAT_NBUF:
                for c in range(C):
                    out_copy(ch - GAT_NBUF, c).wait()
            tpose(ch)
            for c in range(C):
                out_copy(ch, c).start()
            if ch + GAT_NBUF < n_ch:
                idx_copy(ch + GAT_NBUF).start()
        for ch in range(max(0, n_ch - GAT_NBUF), n_ch):
            for c in range(C):
                out_copy(ch, c).wait()

    return k(idx_flat, te)


DEC_B = 16384
DEC_L = 50


def _decode_body(w_ref, b_ref, *refs):
    g_refs, o_ref = refs[:C], refs[C]
    rows = [g_refs[c][...].reshape(1, DEC_B) for c in range(C)]
    gblk = jnp.concatenate(rows, axis=0)
    acc = jnp.dot(w_ref[...], gblk, preferred_element_type=jnp.float32)
    o_ref[...] = jnp.maximum(acc + b_ref[...], 0.0).reshape(1, D, DEC_B)


def _decode(dec_wT, dec_b, g_cols):
    return pl.pallas_call(
        _decode_body,
        grid=(DEC_L,),
        in_specs=[pl.BlockSpec((D, C), lambda l: (0, 0)),
                  pl.BlockSpec((D, 1), lambda l: (0, 0))] +
                 [pl.BlockSpec((DEC_B,), lambda l: (l,)) for _ in range(C)],
        out_specs=pl.BlockSpec((1, D, DEC_B), lambda l: (l, 0, 0)),
        out_shape=jax.ShapeDtypeStruct((DEC_L, D, DEC_B), jnp.float32),
    )(dec_wT, dec_b.reshape(D, 1), *g_cols)


def kernel(input_x, table, enc_w, enc_b, dec_w, dec_b):
    B, L = input_x.shape
    idx_flat = input_x.T.reshape(-1).astype(jnp.int32)
    tableT = table.T
    te_cols = _encode_table(tableT, enc_w.T, enc_b)
    te = _sc_pack(te_cols)
    g_cols = _sc_gather_t(te, idx_flat)
    out3d = _decode(dec_w.T, dec_b, g_cols)
    return out3d.transpose(2, 0, 1)


# speedup vs baseline: 2.2906x; 1.0288x over previous
import functools

import jax
import jax.numpy as jnp
from jax import lax
from jax.experimental import pallas as pl
from jax.experimental.pallas import tpu as pltpu
from jax.experimental.pallas import tpu_sc as plsc

V = 1000000
D = 64
C = 16
ENC_BLK = 4096


def _encode_body(t_ref, w_ref, b_ref, *o_refs):
    x = jnp.dot(w_ref[...], t_ref[...], preferred_element_type=jnp.float32)
    x = jnp.maximum(x + b_ref[...], 0.0)
    for c in range(C):
        o_refs[c][...] = x[c, :]


def _encode_table(tableT, enc_wT, enc_b):
    ng = (V + ENC_BLK - 1) // ENC_BLK
    return pl.pallas_call(
        _encode_body,
        grid=(ng,),
        in_specs=[
            pl.BlockSpec((D, ENC_BLK), lambda i: (0, i)),
            pl.BlockSpec((C, D), lambda i: (0, 0)),
            pl.BlockSpec((C, 1), lambda i: (0, 0)),
        ],
        out_specs=[pl.BlockSpec((ENC_BLK,), lambda i: (i,)) for _ in range(C)],
        out_shape=[jax.ShapeDtypeStruct((V,), jnp.float32) for _ in range(C)],
    )(tableT, enc_wT, enc_b.reshape(C, 1))


PACK_CH = 1600
PACK_NCH = V // PACK_CH
PACK_NBUF = 2


def _sc_pack(te_cols):
    info = plsc.get_sparse_core_info()
    NC, NS = info.num_cores, info.num_subcores
    NW = NC * NS
    mesh = plsc.VectorSubcoreMesh(core_axis_name="c", subcore_axis_name="s")
    max_k = (PACK_NCH + NW - 1) // NW

    @functools.partial(
        pl.kernel,
        out_type=jax.ShapeDtypeStruct((V, C), jnp.float32),
        mesh=mesh,
        compiler_params=pltpu.CompilerParams(use_tc_tiling_on_sc=False, needs_layout_passes=False),
        scratch_types=[
            pltpu.VMEM((PACK_NBUF, C, PACK_CH), jnp.float32),
            pltpu.VMEM((PACK_NBUF, PACK_CH, C), jnp.float32),
            pltpu.SemaphoreType.DMA((PACK_NBUF,)),
            pltpu.SemaphoreType.DMA((PACK_NBUF,)),
        ],
    )
    def k(*refs):
        col_hbm = refs[:C]
        te_hbm = refs[C]
        in_v, out_v = refs[C + 1], refs[C + 2]
        is_sem, os_sem = refs[C + 3], refs[C + 4]
        wid = lax.axis_index("s") * NC + lax.axis_index("c")
        lanes = jnp.arange(16, dtype=jnp.int32)

        def in_copy(kk, c):
            b = kk % PACK_NBUF
            g = wid + kk * NW
            return pltpu.make_async_copy(
                col_hbm[c].at[pl.ds(g * PACK_CH, PACK_CH)], in_v.at[b, c],
                is_sem.at[b])

        def out_copy(kk):
            b = kk % PACK_NBUF
            g = wid + kk * NW
            return pltpu.make_async_copy(
                out_v.at[b], te_hbm.at[pl.ds(g * PACK_CH, PACK_CH)],
                os_sem.at[b])

        def valid(kk):
            return wid + kk * NW < PACK_NCH

        def tpose(kk):
            b = kk % PACK_NBUF

            def step(t8, carry):
                for u in range(8):
                    t = t8 * 8 + u
                    col = plsc.load_gather(
                        in_v.at[b], [lanes, jnp.full((16,), t, jnp.int32)])
                    out_v[b, t, :] = col
                return carry

            lax.fori_loop(0, PACK_CH // 8, step, 0)

        @pl.when(valid(0))
        def _():
            for c in range(C):
                in_copy(0, c).start()

        for kk in range(max_k):
            if kk + 1 < max_k:
                @pl.when(valid(kk + 1))
                def _(kk=kk):
                    for c in range(C):
                        in_copy(kk + 1, c).start()

            @pl.when(valid(kk))
            def _(kk=kk):
                for c in range(C):
                    in_copy(kk, c).wait()
                if kk >= PACK_NBUF:
                    out_copy(kk - PACK_NBUF).wait()
                tpose(kk)
                out_copy(kk).start()

        for kk in range(max(0, max_k - PACK_NBUF), max_k):
            @pl.when(valid(kk))
            def _(kk=kk):
                out_copy(kk).wait()

    return k(*te_cols)


GAT_CH = 1600
GAT_NBUF = 2


def _sc_gather_t(te, idx_flat):
    N = idx_flat.shape[0]
    info = plsc.get_sparse_core_info()
    NC, NS = info.num_cores, info.num_subcores
    NW = NC * NS
    assert N % NW == 0
    b_per_w = N // NW
    assert b_per_w % GAT_CH == 0
    n_ch = b_per_w // GAT_CH
    mesh = plsc.VectorSubcoreMesh(core_axis_name="c", subcore_axis_name="s")

    @functools.partial(
        pl.kernel,
        out_type=[jax.ShapeDtypeStruct((N,), jnp.float32) for _ in range(C)],
        mesh=mesh,
        compiler_params=pltpu.CompilerParams(use_tc_tiling_on_sc=False, needs_layout_passes=False),
        scratch_types=[
            pltpu.VMEM((GAT_NBUF, GAT_CH), jnp.int32),
            pltpu.VMEM((GAT_NBUF, GAT_CH, C), jnp.float32),
            pltpu.VMEM((GAT_NBUF, C, GAT_CH), jnp.float32),
            pltpu.SemaphoreType.DMA((GAT_NBUF,)),
            pltpu.SemaphoreType.DMA((GAT_NBUF,)),
            pltpu.SemaphoreType.DMA((GAT_NBUF,)),
        ],
    )
    def k(*refs):
        idx_hbm, te_hbm = refs[0], refs[1]
        g_hbm = refs[2:2 + C]
        idx_v, rows_v, trows_v = refs[2 + C], refs[3 + C], refs[4 + C]
        is_sem, gs_sem, os_sem = refs[5 + C], refs[6 + C], refs[7 + C]
        wid = lax.axis_index("s") * NC + lax.axis_index("c")
        base = wid * b_per_w
        lanes = jnp.arange(16, dtype=jnp.int32)

        def idx_copy(ch):
            b = ch % GAT_NBUF
            return pltpu.make_async_copy(
                idx_hbm.at[pl.ds(base + ch * GAT_CH, GAT_CH)], idx_v.at[b],
                is_sem.at[b])

        def gat_copy(ch):
            b = ch % GAT_NBUF
            return pltpu.make_async_copy(
                te_hbm.at[idx_v.at[b]], rows_v.at[b], gs_sem.at[b])

        def out_copy(ch, c):
            b = ch % GAT_NBUF
            return pltpu.make_async_copy(
                trows_v.at[b, c], g_hbm[c].at[pl.ds(base + ch * GAT_CH, GAT_CH)],
                os_sem.at[b])

        def tpose(ch):
            b = ch % GAT_NBUF

            def step(t8, carry):
                for u in range(8):
                    t = t8 * 8 + u
                    row = rows_v[b, t, :]
                    plsc.store_scatter(
                        trows_v.at[b], [lanes, jnp.full((16,), t, jnp.int32)], row)
                return carry

            lax.fori_loop(0, GAT_CH // 8, step, 0)

        idx_copy(0).start()
        idx_copy(1).start()
        idx_copy(0).wait()
        gat_copy(0).start()

        for ch in range(n_ch):
            gat_copy(ch).wait()
            if ch + 1 < n_ch:
                idx_copy(ch + 1).wait()
                gat_copy(ch + 1).start()
            if ch >= GAT_NBUF:
                for c in range(C):
                    out_copy(ch - GAT_NBUF, c).wait()
            tpose(ch)
            for c in range(C):
                out_copy(ch, c).start()
            if ch + GAT_NBUF < n_ch:
                idx_copy(ch + GAT_NBUF).start()
        for ch in range(max(0, n_ch - GAT_NBUF), n_ch):
            for c in range(C):
                out_copy(ch, c).wait()

    return k(idx_flat, te)


DEC_B = 16384
DEC_L = 50


def _decode_body(w_ref, b_ref, *refs):
    g_refs, o_ref = refs[:C], refs[C]
    rows = [g_refs[c][...].reshape(1, DEC_B) for c in range(C)]
    gblk = jnp.concatenate(rows, axis=0)
    acc = jnp.dot(w_ref[...], gblk, preferred_element_type=jnp.float32)
    o_ref[...] = jnp.maximum(acc + b_ref[...], 0.0).reshape(1, D, DEC_B)


def _decode(dec_wT, dec_b, g_cols):
    return pl.pallas_call(
        _decode_body,
        grid=(DEC_L,),
        in_specs=[pl.BlockSpec((D, C), lambda l: (0, 0)),
                  pl.BlockSpec((D, 1), lambda l: (0, 0))] +
                 [pl.BlockSpec((DEC_B,), lambda l: (l,)) for _ in range(C)],
        out_specs=pl.BlockSpec((1, D, DEC_B), lambda l: (l, 0, 0)),
        out_shape=jax.ShapeDtypeStruct((DEC_L, D, DEC_B), jnp.float32),
    )(dec_wT, dec_b.reshape(D, 1), *g_cols)


def kernel(input_x, table, enc_w, enc_b, dec_w, dec_b):
    B, L = input_x.shape
    idx_flat = input_x.T.reshape(-1).astype(jnp.int32)
    tableT = table.T
    te_cols = _encode_table(tableT, enc_w.T, enc_b)
    te = _sc_pack(te_cols)
    g_cols = _sc_gather_t(te, idx_flat)
    out3d = _decode(dec_w.T, dec_b, g_cols)
    return out3d.transpose(2, 0, 1)


# speedup vs baseline: 2.9578x; 1.2913x over previous
import functools

import jax
import jax.numpy as jnp
from jax import lax
from jax.experimental import pallas as pl
from jax.experimental.pallas import tpu as pltpu
from jax.experimental.pallas import tpu_sc as plsc

V = 1000000
D = 64
C = 16
ENC_BLK = 4096


def _encode_body(t_ref, w_ref, b_ref, *o_refs):
    x = jnp.dot(w_ref[...], t_ref[...], preferred_element_type=jnp.float32)
    x = jnp.maximum(x + b_ref[...], 0.0)
    for c in range(C):
        o_refs[c][...] = x[c, :]


def _encode_table(tableT, enc_wT, enc_b):
    ng = (V + ENC_BLK - 1) // ENC_BLK
    return pl.pallas_call(
        _encode_body,
        grid=(ng,),
        in_specs=[
            pl.BlockSpec((D, ENC_BLK), lambda i: (0, i)),
            pl.BlockSpec((C, D), lambda i: (0, 0)),
            pl.BlockSpec((C, 1), lambda i: (0, 0)),
        ],
        out_specs=[pl.BlockSpec((ENC_BLK,), lambda i: (i,)) for _ in range(C)],
        out_shape=[jax.ShapeDtypeStruct((V,), jnp.float32) for _ in range(C)],
    )(tableT, enc_wT, enc_b.reshape(C, 1))


PACK_CH = 1600
PACK_NCH = V // PACK_CH
PACK_NBUF = 2


def _sc_pack(te_cols):
    info = plsc.get_sparse_core_info()
    NC, NS = info.num_cores, info.num_subcores
    NW = NC * NS
    mesh = plsc.VectorSubcoreMesh(core_axis_name="c", subcore_axis_name="s")
    max_k = (PACK_NCH + NW - 1) // NW

    @functools.partial(
        pl.kernel,
        out_type=jax.ShapeDtypeStruct((V, C), jnp.float32),
        mesh=mesh,
        compiler_params=pltpu.CompilerParams(use_tc_tiling_on_sc=False, needs_layout_passes=False),
        scratch_types=[
            pltpu.VMEM((PACK_NBUF, C, PACK_CH), jnp.float32),
            pltpu.VMEM((PACK_NBUF, PACK_CH, C), jnp.float32),
            pltpu.SemaphoreType.DMA((PACK_NBUF,)),
            pltpu.SemaphoreType.DMA((PACK_NBUF,)),
        ],
    )
    def k(*refs):
        col_hbm = refs[:C]
        te_hbm = refs[C]
        in_v, out_v = refs[C + 1], refs[C + 2]
        is_sem, os_sem = refs[C + 3], refs[C + 4]
        wid = lax.axis_index("s") * NC + lax.axis_index("c")
        lanes = jnp.arange(16, dtype=jnp.int32)

        def in_copy(kk, c):
            b = kk % PACK_NBUF
            g = wid + kk * NW
            return pltpu.make_async_copy(
                col_hbm[c].at[pl.ds(g * PACK_CH, PACK_CH)], in_v.at[b, c],
                is_sem.at[b])

        def out_copy(kk):
            b = kk % PACK_NBUF
            g = wid + kk * NW
            return pltpu.make_async_copy(
                out_v.at[b], te_hbm.at[pl.ds(g * PACK_CH, PACK_CH)],
                os_sem.at[b])

        def valid(kk):
            return wid + kk * NW < PACK_NCH

        def tpose(kk):
            b = kk % PACK_NBUF

            @plsc.parallel_loop(0, PACK_CH, step=1, unroll=8)
            def _(t):
                col = plsc.load_gather(
                    in_v.at[b], [lanes, jnp.full((16,), t, jnp.int32)])
                out_v[b, t, :] = col

        @pl.when(valid(0))
        def _():
            for c in range(C):
                in_copy(0, c).start()

        for kk in range(max_k):
            if kk + 1 < max_k:
                @pl.when(valid(kk + 1))
                def _(kk=kk):
                    for c in range(C):
                        in_copy(kk + 1, c).start()

            @pl.when(valid(kk))
            def _(kk=kk):
                for c in range(C):
                    in_copy(kk, c).wait()
                if kk >= PACK_NBUF:
                    out_copy(kk - PACK_NBUF).wait()
                tpose(kk)
                out_copy(kk).start()

        for kk in range(max(0, max_k - PACK_NBUF), max_k):
            @pl.when(valid(kk))
            def _(kk=kk):
                out_copy(kk).wait()

    return k(*te_cols)


GAT_CH = 1600
GAT_NBUF = 2


def _sc_gather_t(te, idx_flat):
    N = idx_flat.shape[0]
    info = plsc.get_sparse_core_info()
    NC, NS = info.num_cores, info.num_subcores
    NW = NC * NS
    assert N % NW == 0
    b_per_w = N // NW
    assert b_per_w % GAT_CH == 0
    n_ch = b_per_w // GAT_CH
    mesh = plsc.VectorSubcoreMesh(core_axis_name="c", subcore_axis_name="s")

    @functools.partial(
        pl.kernel,
        out_type=[jax.ShapeDtypeStruct((N,), jnp.float32) for _ in range(C)],
        mesh=mesh,
        compiler_params=pltpu.CompilerParams(use_tc_tiling_on_sc=False, needs_layout_passes=False),
        scratch_types=[
            pltpu.VMEM((GAT_NBUF, GAT_CH), jnp.int32),
            pltpu.VMEM((GAT_NBUF, GAT_CH, C), jnp.float32),
            pltpu.VMEM((GAT_NBUF, C, GAT_CH), jnp.float32),
            pltpu.SemaphoreType.DMA((GAT_NBUF,)),
            pltpu.SemaphoreType.DMA((GAT_NBUF,)),
            pltpu.SemaphoreType.DMA((GAT_NBUF,)),
        ],
    )
    def k(*refs):
        idx_hbm, te_hbm = refs[0], refs[1]
        g_hbm = refs[2:2 + C]
        idx_v, rows_v, trows_v = refs[2 + C], refs[3 + C], refs[4 + C]
        is_sem, gs_sem, os_sem = refs[5 + C], refs[6 + C], refs[7 + C]
        wid = lax.axis_index("s") * NC + lax.axis_index("c")
        base = wid * b_per_w
        lanes = jnp.arange(16, dtype=jnp.int32)

        def idx_copy(ch):
            b = ch % GAT_NBUF
            return pltpu.make_async_copy(
                idx_hbm.at[pl.ds(base + ch * GAT_CH, GAT_CH)], idx_v.at[b],
                is_sem.at[b])

        def gat_copy(ch):
            b = ch % GAT_NBUF
            return pltpu.make_async_copy(
                te_hbm.at[idx_v.at[b]], rows_v.at[b], gs_sem.at[b])

        def out_copy(ch, c):
            b = ch % GAT_NBUF
            return pltpu.make_async_copy(
                trows_v.at[b, c], g_hbm[c].at[pl.ds(base + ch * GAT_CH, GAT_CH)],
                os_sem.at[b])

        def tpose(ch):
            b = ch % GAT_NBUF

            @plsc.parallel_loop(0, GAT_CH, step=1, unroll=8)
            def _(t):
                row = rows_v[b, t, :]
                plsc.store_scatter(
                    trows_v.at[b], [lanes, jnp.full((16,), t, jnp.int32)], row)

        idx_copy(0).start()
        idx_copy(1).start()
        idx_copy(0).wait()
        gat_copy(0).start()

        for ch in range(n_ch):
            gat_copy(ch).wait()
            if ch + 1 < n_ch:
                idx_copy(ch + 1).wait()
                gat_copy(ch + 1).start()
            if ch >= GAT_NBUF:
                for c in range(C):
                    out_copy(ch - GAT_NBUF, c).wait()
            tpose(ch)
            for c in range(C):
                out_copy(ch, c).start()
            if ch + GAT_NBUF < n_ch:
                idx_copy(ch + GAT_NBUF).start()
        for ch in range(max(0, n_ch - GAT_NBUF), n_ch):
            for c in range(C):
                out_copy(ch, c).wait()

    return k(idx_flat, te)


DEC_B = 16384
DEC_L = 50


def _decode_body(w_ref, b_ref, *refs):
    g_refs, o_ref = refs[:C], refs[C]
    rows = [g_refs[c][...].reshape(1, DEC_B) for c in range(C)]
    gblk = jnp.concatenate(rows, axis=0)
    acc = jnp.dot(w_ref[...], gblk, preferred_element_type=jnp.float32)
    o_ref[...] = jnp.maximum(acc + b_ref[...], 0.0).reshape(1, D, DEC_B)


def _decode(dec_wT, dec_b, g_cols):
    return pl.pallas_call(
        _decode_body,
        grid=(DEC_L,),
        in_specs=[pl.BlockSpec((D, C), lambda l: (0, 0)),
                  pl.BlockSpec((D, 1), lambda l: (0, 0))] +
                 [pl.BlockSpec((DEC_B,), lambda l: (l,)) for _ in range(C)],
        out_specs=pl.BlockSpec((1, D, DEC_B), lambda l: (l, 0, 0)),
        out_shape=jax.ShapeDtypeStruct((DEC_L, D, DEC_B), jnp.float32),
    )(dec_wT, dec_b.reshape(D, 1), *g_cols)


def kernel(input_x, table, enc_w, enc_b, dec_w, dec_b):
    B, L = input_x.shape
    idx_flat = input_x.T.reshape(-1).astype(jnp.int32)
    tableT = table.T
    te_cols = _encode_table(tableT, enc_w.T, enc_b)
    te = _sc_pack(te_cols)
    g_cols = _sc_gather_t(te, idx_flat)
    out3d = _decode(dec_w.T, dec_b, g_cols)
    return out3d.transpose(2, 0, 1)


# speedup vs baseline: 3.2386x; 1.0949x over previous
import functools

import jax
import jax.numpy as jnp
from jax import lax
from jax.experimental import pallas as pl
from jax.experimental.pallas import tpu as pltpu
from jax.experimental.pallas import tpu_sc as plsc

V = 1000000
D = 64
C = 16
ENC_BLK = 4096


def _encode_body(t_ref, w_ref, b_ref, *o_refs):
    x = jnp.dot(w_ref[...], t_ref[...], preferred_element_type=jnp.float32)
    x = jnp.maximum(x + b_ref[...], 0.0)
    for c in range(C):
        o_refs[c][...] = x[c, :]


def _encode_table(tableT, enc_wT, enc_b):
    ng = (V + ENC_BLK - 1) // ENC_BLK
    return pl.pallas_call(
        _encode_body,
        grid=(ng,),
        in_specs=[
            pl.BlockSpec((D, ENC_BLK), lambda i: (0, i)),
            pl.BlockSpec((C, D), lambda i: (0, 0)),
            pl.BlockSpec((C, 1), lambda i: (0, 0)),
        ],
        out_specs=[pl.BlockSpec((ENC_BLK,), lambda i: (i,)) for _ in range(C)],
        out_shape=[jax.ShapeDtypeStruct((V,), jnp.float32) for _ in range(C)],
    )(tableT, enc_wT, enc_b.reshape(C, 1))


PACK_CH = 1600
PACK_NCH = V // PACK_CH
PACK_NBUF = 2


def _sc_pack(te_cols):
    info = plsc.get_sparse_core_info()
    NC, NS = info.num_cores, info.num_subcores
    NW = NC * NS
    mesh = plsc.VectorSubcoreMesh(core_axis_name="c", subcore_axis_name="s")
    max_k = (PACK_NCH + NW - 1) // NW

    @functools.partial(
        pl.kernel,
        out_type=jax.ShapeDtypeStruct((V, C), jnp.float32),
        mesh=mesh,
        compiler_params=pltpu.CompilerParams(use_tc_tiling_on_sc=False, needs_layout_passes=False),
        scratch_types=[
            pltpu.VMEM((PACK_NBUF, C * PACK_CH), jnp.float32),
            pltpu.VMEM((PACK_NBUF, PACK_CH, C), jnp.float32),
            pltpu.SemaphoreType.DMA((PACK_NBUF,)),
            pltpu.SemaphoreType.DMA((PACK_NBUF,)),
        ],
    )
    def k(*refs):
        col_hbm = refs[:C]
        te_hbm = refs[C]
        in_v, out_v = refs[C + 1], refs[C + 2]
        is_sem, os_sem = refs[C + 3], refs[C + 4]
        wid = lax.axis_index("s") * NC + lax.axis_index("c")
        lanes = jnp.arange(16, dtype=jnp.int32)

        def in_copy(kk, c):
            b = kk % PACK_NBUF
            g = wid + kk * NW
            return pltpu.make_async_copy(
                col_hbm[c].at[pl.ds(g * PACK_CH, PACK_CH)],
                in_v.at[b, pl.ds(c * PACK_CH, PACK_CH)], is_sem.at[b])

        def out_copy(kk):
            b = kk % PACK_NBUF
            g = wid + kk * NW
            return pltpu.make_async_copy(
                out_v.at[b], te_hbm.at[pl.ds(g * PACK_CH, PACK_CH)],
                os_sem.at[b])

        def valid(kk):
            return wid + kk * NW < PACK_NCH

        lanes_scaled = lanes * PACK_CH

        def tpose(kk):
            b = kk % PACK_NBUF

            @plsc.parallel_loop(0, PACK_CH, step=1, unroll=8)
            def _(t):
                col = plsc.load_gather(in_v.at[b], [lanes_scaled + t])
                out_v[b, t, :] = col

        @pl.when(valid(0))
        def _():
            for c in range(C):
                in_copy(0, c).start()

        for kk in range(max_k):
            if kk + 1 < max_k:
                @pl.when(valid(kk + 1))
                def _(kk=kk):
                    for c in range(C):
                        in_copy(kk + 1, c).start()

            @pl.when(valid(kk))
            def _(kk=kk):
                for c in range(C):
                    in_copy(kk, c).wait()
                if kk >= PACK_NBUF:
                    out_copy(kk - PACK_NBUF).wait()
                tpose(kk)
                out_copy(kk).start()

        for kk in range(max(0, max_k - PACK_NBUF), max_k):
            @pl.when(valid(kk))
            def _(kk=kk):
                out_copy(kk).wait()

    return k(*te_cols)


GAT_CH = 1600
GAT_NBUF = 2


def _sc_gather_t(te, idx_flat, off0, N):
    info = plsc.get_sparse_core_info()
    NC, NS = info.num_cores, info.num_subcores
    NW = NC * NS
    assert N % NW == 0
    b_per_w = N // NW
    assert b_per_w % GAT_CH == 0
    n_ch = b_per_w // GAT_CH
    mesh = plsc.VectorSubcoreMesh(core_axis_name="c", subcore_axis_name="s")

    @functools.partial(
        pl.kernel,
        out_type=[jax.ShapeDtypeStruct((N,), jnp.float32) for _ in range(C)],
        mesh=mesh,
        compiler_params=pltpu.CompilerParams(use_tc_tiling_on_sc=False, needs_layout_passes=False),
        scratch_types=[
            pltpu.VMEM((GAT_NBUF, GAT_CH), jnp.int32),
            pltpu.VMEM((GAT_NBUF, GAT_CH, C), jnp.float32),
            pltpu.VMEM((GAT_NBUF, C * GAT_CH), jnp.float32),
            pltpu.SemaphoreType.DMA((GAT_NBUF,)),
            pltpu.SemaphoreType.DMA((GAT_NBUF,)),
            pltpu.SemaphoreType.DMA((GAT_NBUF,)),
        ],
    )
    def k(*refs):
        idx_hbm, te_hbm = refs[0], refs[1]
        g_hbm = refs[2:2 + C]
        idx_v, rows_v, trows_v = refs[2 + C], refs[3 + C], refs[4 + C]
        is_sem, gs_sem, os_sem = refs[5 + C], refs[6 + C], refs[7 + C]
        wid = lax.axis_index("s") * NC + lax.axis_index("c")
        base = wid * b_per_w
        lanes = jnp.arange(16, dtype=jnp.int32)

        def idx_copy(ch):
            b = ch % GAT_NBUF
            return pltpu.make_async_copy(
                idx_hbm.at[pl.ds(off0 + base + ch * GAT_CH, GAT_CH)], idx_v.at[b],
                is_sem.at[b])

        def gat_copy(ch):
            b = ch % GAT_NBUF
            return pltpu.make_async_copy(
                te_hbm.at[idx_v.at[b]], rows_v.at[b], gs_sem.at[b])

        def out_copy(ch, c):
            b = ch % GAT_NBUF
            return pltpu.make_async_copy(
                trows_v.at[b, pl.ds(c * GAT_CH, GAT_CH)],
                g_hbm[c].at[pl.ds(base + ch * GAT_CH, GAT_CH)], os_sem.at[b])

        lanes_scaled = lanes * GAT_CH

        def tpose(ch):
            b = ch % GAT_NBUF

            @plsc.parallel_loop(0, GAT_CH, step=1, unroll=8)
            def _(t):
                row = rows_v[b, t, :]
                plsc.store_scatter(trows_v.at[b], [lanes_scaled + t], row)

        idx_copy(0).start()
        idx_copy(1).start()
        idx_copy(0).wait()
        gat_copy(0).start()

        for ch in range(n_ch):
            gat_copy(ch).wait()
            if ch + 1 < n_ch:
                idx_copy(ch + 1).wait()
                gat_copy(ch + 1).start()
            if ch >= GAT_NBUF:
                for c in range(C):
                    out_copy(ch - GAT_NBUF, c).wait()
            tpose(ch)
            for c in range(C):
                out_copy(ch, c).start()
            if ch + GAT_NBUF < n_ch:
                idx_copy(ch + GAT_NBUF).start()
        for ch in range(max(0, n_ch - GAT_NBUF), n_ch):
            for c in range(C):
                out_copy(ch, c).wait()

    return k(idx_flat, te)


DEC_B = 16384
DEC_L = 50


def _decode_body(w_ref, b_ref, *refs):
    g_refs, o_ref = refs[:C], refs[C]
    rows = [g_refs[c][...].reshape(1, DEC_B) for c in range(C)]
    gblk = jnp.concatenate(rows, axis=0)
    acc = jnp.dot(w_ref[...], gblk, preferred_element_type=jnp.float32)
    o_ref[...] = jnp.maximum(acc + b_ref[...], 0.0).reshape(1, D, DEC_B)


def _decode_body2(prev_ref, w_ref, b_ref, *refs):
    del prev_ref
    _decode_body(w_ref, b_ref, *refs)


def _decode_first(dec_wT, dec_b, g_cols, nl):
    return pl.pallas_call(
        _decode_body,
        grid=(nl,),
        in_specs=[pl.BlockSpec((D, C), lambda l: (0, 0)),
                  pl.BlockSpec((D, 1), lambda l: (0, 0))] +
                 [pl.BlockSpec((DEC_B,), lambda l: (l,)) for _ in range(C)],
        out_specs=pl.BlockSpec((1, D, DEC_B), lambda l: (l, 0, 0)),
        out_shape=jax.ShapeDtypeStruct((DEC_L, D, DEC_B), jnp.float32),
    )(dec_wT, dec_b.reshape(D, 1), *g_cols)


def _decode_second(prev, dec_wT, dec_b, g_cols, l0, nl):
    return pl.pallas_call(
        _decode_body2,
        grid=(nl,),
        in_specs=[pl.BlockSpec(memory_space=pl.ANY),
                  pl.BlockSpec((D, C), lambda l: (0, 0)),
                  pl.BlockSpec((D, 1), lambda l: (0, 0))] +
                 [pl.BlockSpec((DEC_B,), lambda l: (l,)) for _ in range(C)],
        out_specs=pl.BlockSpec((1, D, DEC_B), lambda l: (l + l0, 0, 0)),
        out_shape=jax.ShapeDtypeStruct((DEC_L, D, DEC_B), jnp.float32),
        input_output_aliases={0: 0},
    )(prev, dec_wT, dec_b.reshape(D, 1), *g_cols)


def kernel(input_x, table, enc_w, enc_b, dec_w, dec_b):
    B, L = input_x.shape
    idx_flat = input_x.T.reshape(-1).astype(jnp.int32)
    tableT = table.T
    te_cols = _encode_table(tableT, enc_w.T, enc_b)
    te = _sc_pack(te_cols)
    N = B * L
    l_half = L // 2
    n_half = l_half * B
    g1 = _sc_gather_t(te, idx_flat, 0, n_half)
    g2 = _sc_gather_t(te, idx_flat, n_half, N - n_half)
    dec_wT = dec_w.T
    out1 = _decode_first(dec_wT, dec_b, g1, l_half)
    out3d = _decode_second(out1, dec_wT, dec_b, g2, l_half, L - l_half)
    return out3d.transpose(2, 0, 1)
